# Initial kernel scaffold; baseline (speedup 1.0000x reference)
import jax
import jax.numpy as jnp
from jax.experimental import pallas as pl


def kernel(logits, uniform):
    raise NotImplementedError("write your pallas kernel here")


# speedup vs baseline: 16.3460x; 16.3460x over previous
import jax
import jax.numpy as jnp
from jax.experimental import pallas as pl

_N = 4096
_K = 64
_ROWS = 256


def _router_block(logits_ref, uniform_ref, out_ref):
    i = pl.program_id(0)
    x = logits_ref[...]
    u = uniform_ref[...]

    noise = -jnp.log(-jnp.log(u + 1e-9) + 1e-9)

    rows = jax.lax.broadcasted_iota(jnp.int32, x.shape, 0) + i * _ROWS
    cols = jax.lax.broadcasted_iota(jnp.int32, x.shape, 1)
    masked = jnp.where(rows == cols, -jnp.inf, x)
    noisy = masked + noise

    bits = jax.lax.bitcast_convert_type(noisy, jnp.uint32)
    neg = bits >= jnp.uint32(0x80000000)
    key = jnp.where(neg, ~bits, bits | jnp.uint32(0x80000000))

    t = jnp.zeros((x.shape[0], 1), jnp.uint32)
    for bit in range(31, -1, -1):
        cand = t | jnp.uint32(1 << bit)
        cnt = jnp.sum((key >= cand).astype(jnp.int32), axis=1, keepdims=True)
        t = jnp.where(cnt >= _K, cand, t)

    out_ref[...] = (key >= t).astype(jnp.float32)


def kernel(logits, uniform):
    grid = _N // _ROWS
    return pl.pallas_call(
        _router_block,
        grid=(grid,),
        in_specs=[
            pl.BlockSpec((_ROWS, _N), lambda i: (i, 0)),
            pl.BlockSpec((_ROWS, _N), lambda i: (i, 0)),
        ],
        out_specs=pl.BlockSpec((_ROWS, _N), lambda i: (i, 0)),
        out_shape=jax.ShapeDtypeStruct((_N, _N), jnp.float32),
    )(logits, uniform)


# speedup vs baseline: 24.5586x; 1.5024x over previous
import jax
import jax.numpy as jnp
from jax.experimental import pallas as pl

_N = 4096
_K = 64
_ROWS = 256


def _router_block(logits_ref, uniform_ref, out_ref):
    i = pl.program_id(0)
    x = logits_ref[...]
    u = uniform_ref[...]

    noise = -jnp.log(-jnp.log(u + 1e-9) + 1e-9)

    rows = jax.lax.broadcasted_iota(jnp.int32, x.shape, 0) + i * _ROWS
    cols = jax.lax.broadcasted_iota(jnp.int32, x.shape, 1)
    masked = jnp.where(rows == cols, -jnp.inf, x)
    noisy = masked + noise

    bits = jax.lax.bitcast_convert_type(noisy, jnp.uint32)
    neg = bits >= jnp.uint32(0x80000000)
    key = jnp.where(neg, ~bits, bits | jnp.uint32(0x80000000))

    hi = (
        (jax.lax.shift_right_logical(key, jnp.uint32(16)) ^ jnp.uint32(0x8000))
        .astype(jnp.uint16)
        .view(jnp.int16)
    )
    th = jnp.zeros((x.shape[0], 1), jnp.uint32)
    for bit in range(15, -1, -1):
        cand = th | jnp.uint32(1 << bit)
        cand16 = (cand ^ jnp.uint32(0x8000)).astype(jnp.uint16).view(jnp.int16)
        acc = (hi >= cand16).astype(jnp.int16)
        w = acc.shape[1] // 2
        while w >= 256:
            acc = acc[:, :w] + acc[:, w:]
            w //= 2
        cnt = jnp.sum(acc.astype(jnp.int32), axis=1, keepdims=True)
        th = jnp.where(cnt >= _K, cand, th)

    th16 = (th ^ jnp.uint32(0x8000)).astype(jnp.uint16).view(jnp.int16)
    lo = (
        ((key & jnp.uint32(0xFFFF)) ^ jnp.uint32(0x8000))
        .astype(jnp.uint16)
        .view(jnp.int16)
    )
    lo_eff = jnp.where(
        hi == th16,
        lo,
        jnp.where(hi > th16, jnp.int16(0x7FFF), jnp.int16(-0x8000)),
    )
    tl = jnp.zeros((x.shape[0], 1), jnp.uint32)
    for bit in range(15, -1, -1):
        cand = tl | jnp.uint32(1 << bit)
        cand16 = (cand ^ jnp.uint32(0x8000)).astype(jnp.uint16).view(jnp.int16)
        acc = (lo_eff >= cand16).astype(jnp.int16)
        w = acc.shape[1] // 2
        while w >= 256:
            acc = acc[:, :w] + acc[:, w:]
            w //= 2
        cnt = jnp.sum(acc.astype(jnp.int32), axis=1, keepdims=True)
        tl = jnp.where(cnt >= _K, cand, tl)

    t = jax.lax.shift_left(th, jnp.uint32(16)) | tl

    out_ref[...] = (key >= t).astype(jnp.float32)


def kernel(logits, uniform):
    grid = _N // _ROWS
    return pl.pallas_call(
        _router_block,
        grid=(grid,),
        in_specs=[
            pl.BlockSpec((_ROWS, _N), lambda i: (i, 0)),
            pl.BlockSpec((_ROWS, _N), lambda i: (i, 0)),
        ],
        out_specs=pl.BlockSpec((_ROWS, _N), lambda i: (i, 0)),
        out_shape=jax.ShapeDtypeStruct((_N, _N), jnp.float32),
    )(logits, uniform)


# speedup vs baseline: 25.0766x; 1.0211x over previous
import jax
import jax.numpy as jnp
from jax.experimental import pallas as pl

_N = 4096
_K = 64
_ROWS = 256


def _router_block(logits_ref, uniform_ref, out_ref):
    i = pl.program_id(0)
    x = logits_ref[...]
    u = uniform_ref[...]

    noise = -jnp.log(-jnp.log(u + 1e-9) + 1e-9)

    rows = jax.lax.broadcasted_iota(jnp.int32, x.shape, 0) + i * _ROWS
    cols = jax.lax.broadcasted_iota(jnp.int32, x.shape, 1)
    masked = jnp.where(rows == cols, -jnp.inf, x)
    noisy = masked + noise

    ibits = jax.lax.bitcast_convert_type(noisy, jnp.int32)
    flip = jax.lax.shift_right_arithmetic(ibits, 31) | jnp.int32(-0x80000000)
    key = (ibits ^ flip).view(jnp.uint32)

    hi = (
        (jax.lax.shift_right_logical(key, jnp.uint32(16)) ^ jnp.uint32(0x8000))
        .astype(jnp.uint16)
        .view(jnp.int16)
    )
    th = jnp.zeros((x.shape[0], 1), jnp.uint32)
    for bit in range(15, -1, -1):
        cand = th | jnp.uint32(1 << bit)
        cand16 = (cand ^ jnp.uint32(0x8000)).astype(jnp.uint16).view(jnp.int16)
        acc = (hi >= cand16).astype(jnp.int16)
        w = acc.shape[1] // 2
        while w >= 256:
            acc = acc[:, :w] + acc[:, w:]
            w //= 2
        cnt = jnp.sum(acc.astype(jnp.int32), axis=1, keepdims=True)
        th = jnp.where(cnt >= _K, cand, th)

    th16 = (th ^ jnp.uint32(0x8000)).astype(jnp.uint16).view(jnp.int16)
    lo = (key.astype(jnp.uint16) ^ jnp.uint16(0x8000)).view(jnp.int16)
    lo_eff = jnp.where(
        hi == th16,
        lo,
        jnp.where(hi > th16, jnp.int16(0x7FFF), jnp.int16(-0x8000)),
    )
    tl = jnp.zeros((x.shape[0], 1), jnp.uint32)
    for bit in range(15, -1, -1):
        cand = tl | jnp.uint32(1 << bit)
        cand16 = (cand ^ jnp.uint32(0x8000)).astype(jnp.uint16).view(jnp.int16)
        acc = (lo_eff >= cand16).astype(jnp.int16)
        w = acc.shape[1] // 2
        while w >= 256:
            acc = acc[:, :w] + acc[:, w:]
            w //= 2
        cnt = jnp.sum(acc.astype(jnp.int32), axis=1, keepdims=True)
        tl = jnp.where(cnt >= _K, cand, tl)

    t = jax.lax.shift_left(th, jnp.uint32(16)) | tl

    out_ref[...] = (key >= t).astype(jnp.float32)


def kernel(logits, uniform):
    grid = _N // _ROWS
    return pl.pallas_call(
        _router_block,
        grid=(grid,),
        in_specs=[
            pl.BlockSpec((_ROWS, _N), lambda i: (i, 0)),
            pl.BlockSpec((_ROWS, _N), lambda i: (i, 0)),
        ],
        out_specs=pl.BlockSpec((_ROWS, _N), lambda i: (i, 0)),
        out_shape=jax.ShapeDtypeStruct((_N, _N), jnp.float32),
    )(logits, uniform)
